# baseline (speedup 1.0000x reference)
import functools

import jax
import jax.numpy as jnp
from jax import lax
from jax.experimental import pallas as pl
from jax.experimental.pallas import tpu as pltpu
from jax.experimental.pallas import tpu_sc as plsc

B, N, F = 4, 2048, 64
KEEP = 1024
BLK = 512
NB = 256


def _score_body(a_ref, x_ref, w_ref, o_ref):
    a = a_ref[0]
    x = x_ref[0]
    w = w_ref[...]
    supT = lax.dot_general(x, a, (((0,), (1,)), ((), ())),
                           preferred_element_type=jnp.float32)
    s = lax.dot_general(supT, w, (((0,), (0,)), ((), ())),
                        preferred_element_type=jnp.float32)
    o_ref[0] = jnp.tanh(s)


def _scores(Xs, As, w):
    return pl.pallas_call(
        _score_body,
        grid=(B, N // BLK),
        in_specs=[
            pl.BlockSpec((1, BLK, N), lambda b, i: (b, i, 0)),
            pl.BlockSpec((1, N, F), lambda b, i: (b, 0, 0)),
            pl.BlockSpec((F, 1), lambda b, i: (0, 0)),
        ],
        out_specs=pl.BlockSpec((1, BLK, 1), lambda b, i: (b, i, 0)),
        out_shape=jax.ShapeDtypeStruct((B, N, 1), jnp.float32),
    )(As, Xs, w)


def _total_order_key(s):
    b = lax.bitcast_convert_type(s, jnp.int32)
    return jnp.where(b >= 0, b, ~b ^ jnp.int32(-2147483648))


def _rank_select_body(sc_ref, sr_ref, kv_ref, ki_ref):
    s_col = sc_ref[0]
    s_row = sr_ref[0]
    k_row = _total_order_key(s_row)
    acc_idx = jnp.zeros((KEEP,), jnp.float32)
    acc_val = jnp.zeros((KEEP,), jnp.float32)
    for nb in range(N // NB):
        scb = s_col[nb * NB:(nb + 1) * NB]
        kcb = _total_order_key(scb)
        gt = jnp.sum((k_row > kcb).astype(jnp.float32), axis=1, keepdims=True)
        jlt = (lax.broadcasted_iota(jnp.int32, (NB, N), 1)
               < lax.broadcasted_iota(jnp.int32, (NB, N), 0) + nb * NB)
        eq = jnp.sum(((k_row == kcb) & jlt).astype(jnp.float32),
                     axis=1, keepdims=True)
        ranks = gt + eq
        r_row = lax.broadcasted_iota(jnp.int32, (NB, KEEP), 1).astype(jnp.float32)
        mask = ranks == r_row
        n_col = (lax.broadcasted_iota(jnp.int32, (NB, KEEP), 0).astype(jnp.float32)
                 + float(nb * NB))
        acc_idx = acc_idx + jnp.sum(jnp.where(mask, n_col, 0.0), axis=0)
        acc_val = acc_val + jnp.sum(jnp.where(mask, scb, 0.0), axis=0)
    kv_ref[0, 0] = acc_val
    ki_ref[0, 0] = acc_idx.astype(jnp.int32)


def _rank_select(s_col, s_row):
    return pl.pallas_call(
        _rank_select_body,
        grid=(B,),
        in_specs=[
            pl.BlockSpec((1, N, 1), lambda b: (b, 0, 0)),
            pl.BlockSpec((1, 1, N), lambda b: (b, 0, 0)),
        ],
        out_specs=[
            pl.BlockSpec((1, 1, KEEP), lambda b: (b, 0, 0)),
            pl.BlockSpec((1, 1, KEEP), lambda b: (b, 0, 0)),
        ],
        out_shape=[
            jax.ShapeDtypeStruct((B, 1, KEEP), jnp.float32),
            jax.ShapeDtypeStruct((B, 1, KEEP), jnp.int32),
        ],
    )(s_col, s_row)


_SC_MESH = plsc.VectorSubcoreMesh(core_axis_name="c", subcore_axis_name="s")
ROWS_PER_WORKER = KEEP // 8
SUB = 32


GRP = 16


@functools.partial(
    pl.kernel,
    mesh=_SC_MESH,
    compiler_params=pltpu.CompilerParams(use_tc_tiling_on_sc=False,
                                         needs_layout_passes=False),
    out_type=[
        jax.ShapeDtypeStruct((B, KEEP, F), jnp.float32),
        jax.ShapeDtypeStruct((B, KEEP, KEEP), jnp.float32),
    ],
    scratch_types=[
        pltpu.VMEM((KEEP,), jnp.int32),
        pltpu.VMEM((ROWS_PER_WORKER,), jnp.int32),
        pltpu.VMEM((ROWS_PER_WORKER, F), jnp.float32),
        pltpu.VMEM((GRP * N,), jnp.float32),
        pltpu.VMEM((GRP, KEEP), jnp.float32),
        pltpu.SemaphoreType.DMA,
    ],
)
def _sc_gather(As_hbm, Xs_hbm, idx_hbm, xout, aout,
               idx_v, idxr_v, xrows_v, row_buf, out_buf, sem):
    c = lax.axis_index("c")
    s = lax.axis_index("s")
    wid = s * 2 + c
    b = wid // 8
    chunk = wid % 8
    base = chunk * ROWS_PER_WORKER
    pltpu.sync_copy(idx_hbm.at[b], idx_v)
    pltpu.sync_copy(idx_hbm.at[b, pl.ds(base, ROWS_PER_WORKER)], idxr_v)
    pltpu.async_copy(Xs_hbm.at[b].at[idxr_v], xrows_v, sem).wait()
    pltpu.sync_copy(xrows_v, xout.at[b, pl.ds(base, ROWS_PER_WORKER)])

    def group_body(g, carry):
        v = idxr_v[pl.ds(g * GRP, GRP)]
        copies = []
        for lane in range(GRP):
            rid = v[lane]
            copies.append(pltpu.make_async_copy(
                As_hbm.at[b, rid], row_buf.at[pl.ds(lane * N, N)], sem))
        for cp in copies:
            cp.start()
        for cp in copies:
            cp.wait()
        for lane in range(GRP):
            for cc in range(KEEP // 16):
                cols = idx_v[pl.ds(cc * 16, 16)] + lane * N
                vals = plsc.load_gather(row_buf, [cols])
                out_buf[lane, pl.ds(cc * 16, 16)] = vals
        pltpu.sync_copy(out_buf, aout.at[b, pl.ds(base + g * GRP, GRP)])
        return carry

    lax.fori_loop(0, ROWS_PER_WORKER // GRP, group_body, 0)


def kernel(Xs, As, attn_kernel):
    s3 = _scores(Xs, As, attn_kernel)
    s_row = s3.reshape(B, 1, N)
    kv3, ki3 = _rank_select(s3, s_row)
    kv = kv3.reshape(B, KEEP)
    ki = ki3.reshape(B, KEEP)
    xo, ao = _sc_gather(As, Xs, ki)
    return (xo, ao, kv)


# speedup vs baseline: 1.3315x; 1.3315x over previous
import functools

import jax
import jax.numpy as jnp
from jax import lax
from jax.experimental import pallas as pl
from jax.experimental.pallas import tpu as pltpu
from jax.experimental.pallas import tpu_sc as plsc

B, N, F = 4, 2048, 64
KEEP = 1024
BLK = 512
NB = 256


def _score_body(a_ref, x_ref, w_ref, o_ref):
    a = a_ref[0]
    x = x_ref[0]
    w = w_ref[...]
    supT = lax.dot_general(x, a, (((0,), (1,)), ((), ())),
                           preferred_element_type=jnp.float32)
    s = lax.dot_general(supT, w, (((0,), (0,)), ((), ())),
                        preferred_element_type=jnp.float32)
    o_ref[0] = jnp.tanh(s)


def _scores(Xs, As, w):
    return pl.pallas_call(
        _score_body,
        grid=(B, N // BLK),
        in_specs=[
            pl.BlockSpec((1, BLK, N), lambda b, i: (b, i, 0)),
            pl.BlockSpec((1, N, F), lambda b, i: (b, 0, 0)),
            pl.BlockSpec((F, 1), lambda b, i: (0, 0)),
        ],
        out_specs=pl.BlockSpec((1, BLK, 1), lambda b, i: (b, i, 0)),
        out_shape=jax.ShapeDtypeStruct((B, N, 1), jnp.float32),
    )(As, Xs, w)


def _total_order_key(s):
    b = lax.bitcast_convert_type(s, jnp.int32)
    return jnp.where(b >= 0, b, ~b ^ jnp.int32(-2147483648))


def _rank_select_body(sc_ref, sr_ref, kv_ref, ki_ref):
    s_col = sc_ref[0]
    s_row = sr_ref[0]
    k_row = _total_order_key(s_row)
    acc_idx = jnp.zeros((KEEP,), jnp.float32)
    acc_val = jnp.zeros((KEEP,), jnp.float32)
    for nb in range(N // NB):
        scb = s_col[nb * NB:(nb + 1) * NB]
        kcb = _total_order_key(scb)
        gt = jnp.sum((k_row > kcb).astype(jnp.float32), axis=1, keepdims=True)
        jlt = (lax.broadcasted_iota(jnp.int32, (NB, N), 1)
               < lax.broadcasted_iota(jnp.int32, (NB, N), 0) + nb * NB)
        eq = jnp.sum(((k_row == kcb) & jlt).astype(jnp.float32),
                     axis=1, keepdims=True)
        ranks = gt + eq
        r_row = lax.broadcasted_iota(jnp.int32, (NB, KEEP), 1).astype(jnp.float32)
        mask = ranks == r_row
        n_col = (lax.broadcasted_iota(jnp.int32, (NB, KEEP), 0).astype(jnp.float32)
                 + float(nb * NB))
        acc_idx = acc_idx + jnp.sum(jnp.where(mask, n_col, 0.0), axis=0)
        acc_val = acc_val + jnp.sum(jnp.where(mask, scb, 0.0), axis=0)
    kv_ref[0, 0] = acc_val
    ki_ref[0, 0] = acc_idx.astype(jnp.int32)


def _rank_select(s_col, s_row):
    return pl.pallas_call(
        _rank_select_body,
        grid=(B,),
        in_specs=[
            pl.BlockSpec((1, N, 1), lambda b: (b, 0, 0)),
            pl.BlockSpec((1, 1, N), lambda b: (b, 0, 0)),
        ],
        out_specs=[
            pl.BlockSpec((1, 1, KEEP), lambda b: (b, 0, 0)),
            pl.BlockSpec((1, 1, KEEP), lambda b: (b, 0, 0)),
        ],
        out_shape=[
            jax.ShapeDtypeStruct((B, 1, KEEP), jnp.float32),
            jax.ShapeDtypeStruct((B, 1, KEEP), jnp.int32),
        ],
    )(s_col, s_row)


_SC_MESH = plsc.VectorSubcoreMesh(core_axis_name="c", subcore_axis_name="s")
ROWS_PER_WORKER = KEEP // 8
SUB = 32


GRP = 16


@functools.partial(
    pl.kernel,
    mesh=_SC_MESH,
    compiler_params=pltpu.CompilerParams(needs_layout_passes=False),
    out_type=[
        jax.ShapeDtypeStruct((B, KEEP, F), jnp.float32),
        jax.ShapeDtypeStruct((B, KEEP, KEEP), jnp.float32),
    ],
    scratch_types=[
        pltpu.VMEM((KEEP,), jnp.int32),
        pltpu.VMEM((ROWS_PER_WORKER,), jnp.int32),
        pltpu.VMEM((ROWS_PER_WORKER, F), jnp.float32),
        pltpu.VMEM((GRP * N,), jnp.float32),
        pltpu.VMEM((GRP, KEEP), jnp.float32),
        pltpu.SemaphoreType.DMA,
    ],
)
def _sc_gather(As_hbm, Xs_hbm, idx_hbm, xout, aout,
               idx_v, idxr_v, xrows_v, row_buf, out_buf, sem):
    c = lax.axis_index("c")
    s = lax.axis_index("s")
    wid = s * 2 + c
    b = wid // 8
    chunk = wid % 8
    base = chunk * ROWS_PER_WORKER
    pltpu.sync_copy(idx_hbm.at[b], idx_v)
    pltpu.sync_copy(idx_hbm.at[b, pl.ds(base, ROWS_PER_WORKER)], idxr_v)

    def group_body(g, carry):
        v = idxr_v[pl.ds(g * GRP, GRP)]
        copies = []
        for lane in range(GRP):
            rid = v[lane]
            copies.append(pltpu.make_async_copy(
                As_hbm.at[b, rid], row_buf.at[pl.ds(lane * N, N)], sem))
            copies.append(pltpu.make_async_copy(
                Xs_hbm.at[b, rid], xrows_v.at[g * GRP + lane], sem))
        for cp in copies:
            cp.start()
        for cp in copies:
            cp.wait()
        for lane in range(GRP):
            for cc in range(KEEP // 16):
                cols = idx_v[pl.ds(cc * 16, 16)] + lane * N
                vals = plsc.load_gather(row_buf, [cols])
                out_buf[lane, pl.ds(cc * 16, 16)] = vals
        pltpu.sync_copy(out_buf, aout.at[b, pl.ds(base + g * GRP, GRP)])
        return carry

    lax.fori_loop(0, ROWS_PER_WORKER // GRP, group_body, 0)
    pltpu.sync_copy(xrows_v, xout.at[b, pl.ds(base, ROWS_PER_WORKER)])


def kernel(Xs, As, attn_kernel):
    s3 = _scores(Xs, As, attn_kernel)
    s_row = s3.reshape(B, 1, N)
    kv3, ki3 = _rank_select(s3, s_row)
    kv = kv3.reshape(B, KEEP)
    ki = ki3.reshape(B, KEEP)
    xo, ao = _sc_gather(As, Xs, ki)
    return (xo, ao, kv)


# speedup vs baseline: 1.6017x; 1.2030x over previous
import functools

import jax
import jax.numpy as jnp
from jax import lax
from jax.experimental import pallas as pl
from jax.experimental.pallas import tpu as pltpu
from jax.experimental.pallas import tpu_sc as plsc

B, N, F = 4, 2048, 64
KEEP = 1024
BLK = 512
NB = 256


def _score_body(a_ref, x_ref, w_ref, o_ref):
    a = a_ref[0]
    x = x_ref[0]
    w = w_ref[...]
    supT = lax.dot_general(x, a, (((0,), (1,)), ((), ())),
                           preferred_element_type=jnp.float32)
    s = lax.dot_general(supT, w, (((0,), (0,)), ((), ())),
                        preferred_element_type=jnp.float32)
    o_ref[0] = jnp.tanh(s)


def _scores(Xs, As, w):
    return pl.pallas_call(
        _score_body,
        grid=(B, N // BLK),
        in_specs=[
            pl.BlockSpec((1, BLK, N), lambda b, i: (b, i, 0)),
            pl.BlockSpec((1, N, F), lambda b, i: (b, 0, 0)),
            pl.BlockSpec((F, 1), lambda b, i: (0, 0)),
        ],
        out_specs=pl.BlockSpec((1, BLK, 1), lambda b, i: (b, i, 0)),
        out_shape=jax.ShapeDtypeStruct((B, N, 1), jnp.float32),
    )(As, Xs, w)


def _total_order_key(s):
    b = lax.bitcast_convert_type(s, jnp.int32)
    return jnp.where(b >= 0, b, ~b ^ jnp.int32(-2147483648))


def _rank_select_body(sc_ref, sr_ref, kv_ref, ki_ref):
    s_col = sc_ref[0]
    s_row = sr_ref[0]
    k_row = _total_order_key(s_row)
    acc_idx = jnp.zeros((KEEP,), jnp.float32)
    acc_val = jnp.zeros((KEEP,), jnp.float32)
    for nb in range(N // NB):
        scb = s_col[nb * NB:(nb + 1) * NB]
        kcb = _total_order_key(scb)
        gt = jnp.sum((k_row > kcb).astype(jnp.float32), axis=1, keepdims=True)
        jlt = (lax.broadcasted_iota(jnp.int32, (NB, N), 1)
               < lax.broadcasted_iota(jnp.int32, (NB, N), 0) + nb * NB)
        eq = jnp.sum(((k_row == kcb) & jlt).astype(jnp.float32),
                     axis=1, keepdims=True)
        ranks = gt + eq
        r_row = lax.broadcasted_iota(jnp.int32, (NB, KEEP), 1).astype(jnp.float32)
        mask = ranks == r_row
        n_col = (lax.broadcasted_iota(jnp.int32, (NB, KEEP), 0).astype(jnp.float32)
                 + float(nb * NB))
        acc_idx = acc_idx + jnp.sum(jnp.where(mask, n_col, 0.0), axis=0)
        acc_val = acc_val + jnp.sum(jnp.where(mask, scb, 0.0), axis=0)
    kv_ref[0, 0] = acc_val
    ki_ref[0, 0] = acc_idx.astype(jnp.int32)


def _rank_select(s_col, s_row):
    return pl.pallas_call(
        _rank_select_body,
        grid=(B,),
        in_specs=[
            pl.BlockSpec((1, N, 1), lambda b: (b, 0, 0)),
            pl.BlockSpec((1, 1, N), lambda b: (b, 0, 0)),
        ],
        out_specs=[
            pl.BlockSpec((1, 1, KEEP), lambda b: (b, 0, 0)),
            pl.BlockSpec((1, 1, KEEP), lambda b: (b, 0, 0)),
        ],
        out_shape=[
            jax.ShapeDtypeStruct((B, 1, KEEP), jnp.float32),
            jax.ShapeDtypeStruct((B, 1, KEEP), jnp.int32),
        ],
    )(s_col, s_row)


_SC_MESH = plsc.VectorSubcoreMesh(core_axis_name="c", subcore_axis_name="s")
ROWS_PER_WORKER = KEEP // 8
SUB = 32


GRP = 16


@functools.partial(
    pl.kernel,
    mesh=_SC_MESH,
    compiler_params=pltpu.CompilerParams(needs_layout_passes=False),
    out_type=[
        jax.ShapeDtypeStruct((B, KEEP, F), jnp.float32),
        jax.ShapeDtypeStruct((B, KEEP, KEEP), jnp.float32),
    ],
    scratch_types=[
        pltpu.VMEM((KEEP,), jnp.int32),
        pltpu.VMEM((ROWS_PER_WORKER,), jnp.int32),
        pltpu.VMEM((ROWS_PER_WORKER, F), jnp.float32),
        pltpu.VMEM((GRP * N,), jnp.float32),
        pltpu.VMEM((GRP * N,), jnp.float32),
        pltpu.VMEM((GRP, KEEP), jnp.float32),
        pltpu.VMEM((GRP, KEEP), jnp.float32),
        pltpu.SemaphoreType.DMA,
        pltpu.SemaphoreType.DMA,
    ],
)
def _sc_gather(As_hbm, Xs_hbm, idx_hbm, xout, aout,
               idx_v, idxr_v, xrows_v, row_a, row_b, out_a, out_b,
               sem_a, sem_b):
    c = lax.axis_index("c")
    s = lax.axis_index("s")
    wid = s * 2 + c
    b = wid // 8
    chunk = wid % 8
    base = chunk * ROWS_PER_WORKER
    pltpu.sync_copy(idx_hbm.at[b], idx_v)
    pltpu.sync_copy(idx_hbm.at[b, pl.ds(base, ROWS_PER_WORKER)], idxr_v)

    NGRP = ROWS_PER_WORKER // GRP

    def fire(g, row_buf, sem):
        v = idxr_v[pl.ds(g * GRP, GRP)]
        copies = []
        for lane in range(GRP):
            rid = v[lane]
            copies.append(pltpu.make_async_copy(
                As_hbm.at[b, rid], row_buf.at[pl.ds(lane * N, N)], sem))
            copies.append(pltpu.make_async_copy(
                Xs_hbm.at[b, rid], xrows_v.at[g * GRP + lane], sem))
        for cp in copies:
            cp.start()
        return copies

    def drain(g, row_buf, sem):
        for lane in range(GRP):
            pltpu.make_async_copy(
                As_hbm.at[b, 0], row_buf.at[pl.ds(lane * N, N)], sem).wait()
            pltpu.make_async_copy(
                Xs_hbm.at[b, 0], xrows_v.at[g * GRP + lane], sem).wait()

    def process(g, row_buf, out_buf):
        for cc in range(KEEP // 16):
            cols = idx_v[pl.ds(cc * 16, 16)]
            for lane in range(GRP):
                vals = plsc.load_gather(
                    row_buf.at[pl.ds(lane * N, N)], [cols])
                out_buf[lane, pl.ds(cc * 16, 16)] = vals
        pltpu.sync_copy(out_buf, aout.at[b, pl.ds(base + g * GRP, GRP)])

    fire(0, row_a, sem_a)

    def pair_body(t, carry):
        g0 = t * 2
        fire(g0 + 1, row_b, sem_b)
        drain(g0, row_a, sem_a)
        process(g0, row_a, out_a)

        @pl.when(g0 + 2 < NGRP)
        def _():
            fire(g0 + 2, row_a, sem_a)

        drain(g0 + 1, row_b, sem_b)
        process(g0 + 1, row_b, out_b)
        return carry

    lax.fori_loop(0, NGRP // 2, pair_body, 0)
    pltpu.sync_copy(xrows_v, xout.at[b, pl.ds(base, ROWS_PER_WORKER)])


def kernel(Xs, As, attn_kernel):
    s3 = _scores(Xs, As, attn_kernel)
    s_row = s3.reshape(B, 1, N)
    kv3, ki3 = _rank_select(s3, s_row)
    kv = kv3.reshape(B, KEEP)
    ki = ki3.reshape(B, KEEP)
    xo, ao = _sc_gather(As, Xs, ki)
    return (xo, ao, kv)
